# Initial kernel scaffold; baseline (speedup 1.0000x reference)
import jax
import jax.numpy as jnp
from jax import lax
from jax.experimental import pallas as pl
from jax.experimental.pallas import tpu as pltpu
from jax.experimental.pallas import tpu_sc as plsc

N = 10000
F = 128
E = 320000
NG = 16

NC = 2
NS = 16
NT = NC * NS

CH = 128
KCH = 79
EPT = KCH * CH
EP = NT * EPT
NP = 10240
RPT = NP // NS
NBK = RPT // CH


def _sc_mesh():
    return plsc.VectorSubcoreMesh(
        core_axis_name="c", subcore_axis_name="s", num_cores=NC, num_subcores=NS
    )


def _deg_partials(ei, ones_row, zeros_row):
    def body(ei_ref, ones_ref, zeros_ref, out_ref, dst_v, ones_v, row_v, deg_sh):
        c = lax.axis_index("c")
        s = lax.axis_index("s")
        tile = c * NS + s
        pltpu.sync_copy(ones_ref, ones_v)
        pltpu.sync_copy(zeros_ref, row_v)
        pltpu.sync_copy(ei_ref.at[1, pl.ds(tile * KCH, KCH)], dst_v)
        pltpu.sync_copy(row_v, deg_sh.at[pl.ds(s * RPT, RPT)])
        plsc.subcore_barrier()

        def step(k, carry):
            pltpu.sync_copy(ones_v, deg_sh.at[dst_v.at[k]], add=True)
            return carry

        lax.fori_loop(0, KCH, step, 0)
        plsc.subcore_barrier()
        pltpu.sync_copy(deg_sh.at[pl.ds(s * RPT, RPT)], row_v)
        pltpu.sync_copy(row_v, out_ref.at[c, pl.ds(s * RPT, RPT)])

    return pl.kernel(
        body,
        out_type=jax.ShapeDtypeStruct((NC, NP), jnp.float32),
        mesh=_sc_mesh(),
        scratch_types=[
            pltpu.VMEM((KCH, CH), jnp.int32),
            pltpu.VMEM((CH,), jnp.float32),
            pltpu.VMEM((RPT,), jnp.float32),
            pltpu.VMEM_SHARED((NP,), jnp.float32),
        ],
    )(ei, ones_row, zeros_row)


def _edge_scatter(ei, g, zeros_tile):
    def body(ei_ref, g_ref, z_ref, out_ref, src_v, dst_v, rows_v, acc_sh):
        c = lax.axis_index("c")
        s = lax.axis_index("s")
        tile = c * NS + s
        pltpu.sync_copy(z_ref, rows_v)
        for b in range(NBK):
            pltpu.sync_copy(rows_v, acc_sh.at[pl.ds(s * RPT + b * CH, CH)])
        pltpu.sync_copy(ei_ref.at[0, pl.ds(tile * KCH, KCH)], src_v)
        pltpu.sync_copy(ei_ref.at[1, pl.ds(tile * KCH, KCH)], dst_v)
        plsc.subcore_barrier()

        def step(k, carry):
            pltpu.sync_copy(g_ref.at[src_v.at[k]], rows_v)
            pltpu.sync_copy(rows_v, acc_sh.at[dst_v.at[k]], add=True)
            return carry

        lax.fori_loop(0, KCH, step, 0)
        plsc.subcore_barrier()

        def outp(b, carry):
            pltpu.sync_copy(acc_sh.at[pl.ds(s * RPT + b * CH, CH)], rows_v)
            pltpu.sync_copy(rows_v, out_ref.at[c, pl.ds(s * RPT + b * CH, CH)])
            return carry

        lax.fori_loop(0, NBK, outp, 0)

    return pl.kernel(
        body,
        out_type=jax.ShapeDtypeStruct((NC, NP, F), jnp.float32),
        mesh=_sc_mesh(),
        scratch_types=[
            pltpu.VMEM((KCH, CH), jnp.int32),
            pltpu.VMEM((KCH, CH), jnp.int32),
            pltpu.VMEM((CH, F), jnp.float32),
            pltpu.VMEM_SHARED((NP, F), jnp.float32),
        ],
    )(ei, g, zeros_tile)


def _dinv_of(degp_ref):
    return lax.rsqrt(degp_ref[0, :] + degp_ref[1, :] + 1.0)


def _tc_first(x, W, degp):
    def body(x_ref, w_ref, degp_ref, g_ref):
        dinv = _dinv_of(degp_ref)
        h = jnp.dot(x_ref[...], w_ref[...], preferred_element_type=jnp.float32)
        g_ref[...] = h * dinv[:, None]

    return pl.pallas_call(
        body,
        out_shape=jax.ShapeDtypeStruct((NP, F), jnp.float32),
    )(x, W, degp)


def _tc_mid(p, g, degp, b0, W1):
    def body(p_ref, g_ref, degp_ref, b_ref, w_ref, out_ref):
        dinv = _dinv_of(degp_ref)
        t = dinv[:, None] * (p_ref[0] + p_ref[1] + g_ref[...]) + b_ref[...][None, :]
        t = jnp.maximum(t, 0.0)
        h = jnp.dot(t, w_ref[...], preferred_element_type=jnp.float32)
        out_ref[...] = h * dinv[:, None]

    return pl.pallas_call(
        body,
        out_shape=jax.ShapeDtypeStruct((NP, F), jnp.float32),
    )(p, g, degp, b0, W1)


def _tc_tail(p, g, degp, b1, batch_p, Wf0, bf0, gamma0, beta0, Wf1, bf1, gamma1, beta1):
    def body(p_ref, g_ref, degp_ref, b_ref, batch_ref, wf0_ref, bf0_ref, ga0_ref,
             be0_ref, wf1_ref, bf1_ref, ga1_ref, be1_ref, out_ref):
        dinv = _dinv_of(degp_ref)
        h = dinv[:, None] * (p_ref[0] + p_ref[1] + g_ref[...]) + b_ref[...][None, :]
        h = jnp.maximum(h, 0.0)
        bt = batch_ref[...]
        cols = []
        for gi in range(NG):
            m = (bt == gi)
            v = jnp.where(m[:, None], h, -jnp.inf)
            cols.append(jnp.max(v, axis=0, keepdims=True))
        pooled = jnp.concatenate(cols, axis=0)
        pooled = jnp.where(jnp.isfinite(pooled), pooled, 0.0)

        z = jnp.dot(pooled, wf0_ref[...], preferred_element_type=jnp.float32)
        z = jnp.maximum(z + bf0_ref[...][None, :], 0.0)
        mu = jnp.mean(z, axis=0, keepdims=True)
        var = jnp.mean((z - mu) ** 2, axis=0, keepdims=True)
        z = (z - mu) * lax.rsqrt(var + 1e-5) * ga0_ref[...][None, :] + be0_ref[...][None, :]
        z = jnp.dot(z, wf1_ref[...], preferred_element_type=jnp.float32)
        z = jnp.maximum(z + bf1_ref[...][None, :], 0.0)
        mu = jnp.mean(z, axis=0, keepdims=True)
        var = jnp.mean((z - mu) ** 2, axis=0, keepdims=True)
        out_ref[...] = (z - mu) * lax.rsqrt(var + 1e-5) * ga1_ref[...][None, :] + be1_ref[...][None, :]

    return pl.pallas_call(
        body,
        out_shape=jax.ShapeDtypeStruct((NG, F), jnp.float32),
    )(p, g, degp, b1, batch_p, Wf0, bf0, gamma0, beta0, Wf1, bf1, gamma1, beta1)


def kernel(x, edge_index, batch, Wg0, bg0, Wg1, bg1, Wf0, bf0, gamma0, beta0,
           Wf1, bf1, gamma1, beta1):
    xp = jnp.zeros((NP, F), jnp.float32).at[:N].set(x)
    pad_e = EP - E
    pad = jnp.stack([
        jnp.zeros((pad_e,), jnp.int32),
        jnp.full((pad_e,), N, jnp.int32),
    ])
    ei = jnp.concatenate([edge_index, pad], axis=1).reshape(2, EP // CH, CH)
    batch_p = jnp.concatenate([batch, jnp.full((NP - N,), NG, jnp.int32)])
    zeros_tile = jnp.zeros((CH, F), jnp.float32)
    ones_row = jnp.ones((CH,), jnp.float32)
    zeros_row = jnp.zeros((RPT,), jnp.float32)

    degp = _deg_partials(ei, ones_row, zeros_row)
    g0 = _tc_first(xp, Wg0, degp)
    p0 = _edge_scatter(ei, g0, zeros_tile)
    g1 = _tc_mid(p0, g0, degp, bg0, Wg1)
    p1 = _edge_scatter(ei, g1, zeros_tile)
    return _tc_tail(p1, g1, degp, bg1, batch_p,
                    Wf0, bf0, gamma0, beta0, Wf1, bf1, gamma1, beta1)


# speedup vs baseline: 8.1458x; 8.1458x over previous
import jax
import jax.numpy as jnp
from jax import lax
from jax.experimental import pallas as pl
from jax.experimental.pallas import tpu as pltpu
from jax.experimental.pallas import tpu_sc as plsc

N = 10000
F = 128
E = 320000
NG = 16

NC = 2
NS = 16
NT = NC * NS

CH = 128
KCH = 80
EPT = KCH * CH
EP = NT * EPT
NP = 10240
RPT = NP // NS
NBK = RPT // CH


def _sc_mesh():
    return plsc.VectorSubcoreMesh(
        core_axis_name="c", subcore_axis_name="s", num_cores=NC, num_subcores=NS
    )


def _deg_partials(ei, ones_row, zeros_row):
    def body(ei_ref, ones_ref, zeros_ref, out_ref, dst_v, ones_v, row_v, deg_sh):
        c = lax.axis_index("c")
        s = lax.axis_index("s")
        tile = c * NS + s
        pltpu.sync_copy(ones_ref, ones_v)
        pltpu.sync_copy(zeros_ref, row_v)
        pltpu.sync_copy(ei_ref.at[1, pl.ds(tile * KCH, KCH)], dst_v)
        pltpu.sync_copy(row_v, deg_sh.at[pl.ds(s * RPT, RPT)])
        plsc.subcore_barrier()

        def step(k, carry):
            pltpu.sync_copy(ones_v, deg_sh.at[dst_v.at[k]], add=True)
            return carry

        lax.fori_loop(0, KCH, step, 0)
        plsc.subcore_barrier()
        pltpu.sync_copy(deg_sh.at[pl.ds(s * RPT, RPT)], row_v)
        pltpu.sync_copy(row_v, out_ref.at[c, pl.ds(s * RPT, RPT)])

    return pl.kernel(
        body,
        out_type=jax.ShapeDtypeStruct((NC, NP), jnp.float32),
        mesh=_sc_mesh(),
        scratch_types=[
            pltpu.VMEM((KCH, CH), jnp.int32),
            pltpu.VMEM((CH,), jnp.float32),
            pltpu.VMEM((RPT,), jnp.float32),
            pltpu.VMEM_SHARED((NP,), jnp.float32),
        ],
    )(ei, ones_row, zeros_row)


def _edge_scatter(ei, g, zeros_tile):
    def body(ei_ref, g_ref, z_ref, out_ref, src_v, dst_v, rows_v, acc_sh):
        c = lax.axis_index("c")
        s = lax.axis_index("s")
        tile = c * NS + s
        pltpu.sync_copy(z_ref, rows_v)
        for b in range(NBK):
            pltpu.sync_copy(rows_v, acc_sh.at[pl.ds(s * RPT + b * CH, CH)])
        pltpu.sync_copy(ei_ref.at[0, pl.ds(tile * KCH, KCH)], src_v)
        pltpu.sync_copy(ei_ref.at[1, pl.ds(tile * KCH, KCH)], dst_v)
        plsc.subcore_barrier()

        def step(k, carry):
            pltpu.sync_copy(g_ref.at[src_v.at[k]], rows_v)
            pltpu.sync_copy(rows_v, acc_sh.at[dst_v.at[k]], add=True)
            return carry

        lax.fori_loop(0, KCH, step, 0)
        plsc.subcore_barrier()

        def outp(b, carry):
            pltpu.sync_copy(acc_sh.at[pl.ds(s * RPT + b * CH, CH)], rows_v)
            pltpu.sync_copy(rows_v, out_ref.at[c, pl.ds(s * RPT + b * CH, CH)])
            return carry

        lax.fori_loop(0, NBK, outp, 0)

    return pl.kernel(
        body,
        out_type=jax.ShapeDtypeStruct((NC, NP, F), jnp.float32),
        mesh=_sc_mesh(),
        scratch_types=[
            pltpu.VMEM((KCH, CH), jnp.int32),
            pltpu.VMEM((KCH, CH), jnp.int32),
            pltpu.VMEM((CH, F), jnp.float32),
            pltpu.VMEM_SHARED((NP, F), jnp.float32),
        ],
    )(ei, g, zeros_tile)


def _dinv_of(degp_ref):
    return lax.rsqrt(degp_ref[0, :] + degp_ref[1, :] + 1.0)


def _tc_first(x, W, degp):
    def body(x_ref, w_ref, degp_ref, g_ref):
        dinv = _dinv_of(degp_ref)
        h = jnp.dot(x_ref[...], w_ref[...], preferred_element_type=jnp.float32)
        g_ref[...] = h * dinv[:, None]

    return pl.pallas_call(
        body,
        out_shape=jax.ShapeDtypeStruct((NP, F), jnp.float32),
    )(x, W, degp)


def _tc_mid(p, g, degp, b0, W1):
    def body(p_ref, g_ref, degp_ref, b_ref, w_ref, out_ref):
        dinv = _dinv_of(degp_ref)
        t = dinv[:, None] * (p_ref[0] + p_ref[1] + g_ref[...]) + b_ref[...][None, :]
        t = jnp.maximum(t, 0.0)
        h = jnp.dot(t, w_ref[...], preferred_element_type=jnp.float32)
        out_ref[...] = h * dinv[:, None]

    return pl.pallas_call(
        body,
        out_shape=jax.ShapeDtypeStruct((NP, F), jnp.float32),
    )(p, g, degp, b0, W1)


def _tc_tail(p, g, degp, b1, batch_p, Wf0, bf0, gamma0, beta0, Wf1, bf1, gamma1, beta1):
    def body(p_ref, g_ref, degp_ref, b_ref, batch_ref, wf0_ref, bf0_ref, ga0_ref,
             be0_ref, wf1_ref, bf1_ref, ga1_ref, be1_ref, out_ref):
        dinv = _dinv_of(degp_ref)
        h = dinv[:, None] * (p_ref[0] + p_ref[1] + g_ref[...]) + b_ref[...][None, :]
        h = jnp.maximum(h, 0.0)
        bt = batch_ref[...]
        cols = []
        for gi in range(NG):
            pen = jnp.where(bt == gi, 0.0, -jnp.inf)
            v = h + pen[:, None]
            cols.append(jnp.max(v, axis=0, keepdims=True))
        pooled = jnp.concatenate(cols, axis=0)
        pooled = jnp.where(jnp.isfinite(pooled), pooled, 0.0)

        z = jnp.dot(pooled, wf0_ref[...], preferred_element_type=jnp.float32)
        z = jnp.maximum(z + bf0_ref[...][None, :], 0.0)
        mu = jnp.mean(z, axis=0, keepdims=True)
        var = jnp.mean((z - mu) ** 2, axis=0, keepdims=True)
        z = (z - mu) * lax.rsqrt(var + 1e-5) * ga0_ref[...][None, :] + be0_ref[...][None, :]
        z = jnp.dot(z, wf1_ref[...], preferred_element_type=jnp.float32)
        z = jnp.maximum(z + bf1_ref[...][None, :], 0.0)
        mu = jnp.mean(z, axis=0, keepdims=True)
        var = jnp.mean((z - mu) ** 2, axis=0, keepdims=True)
        out_ref[...] = (z - mu) * lax.rsqrt(var + 1e-5) * ga1_ref[...][None, :] + be1_ref[...][None, :]

    return pl.pallas_call(
        body,
        out_shape=jax.ShapeDtypeStruct((NG, F), jnp.float32),
    )(p, g, degp, b1, batch_p, Wf0, bf0, gamma0, beta0, Wf1, bf1, gamma1, beta1)


def kernel(x, edge_index, batch, Wg0, bg0, Wg1, bg1, Wf0, bf0, gamma0, beta0,
           Wf1, bf1, gamma1, beta1):
    xp = jnp.zeros((NP, F), jnp.float32).at[:N].set(x)
    pad_e = EP - E
    pad = jnp.stack([
        jnp.zeros((pad_e,), jnp.int32),
        jnp.full((pad_e,), N, jnp.int32),
    ])
    ei = jnp.concatenate([edge_index, pad], axis=1).reshape(2, EP // CH, CH)
    batch_p = jnp.concatenate([batch, jnp.full((NP - N,), NG, jnp.int32)])
    zeros_tile = jnp.zeros((CH, F), jnp.float32)
    ones_row = jnp.ones((CH,), jnp.float32)
    zeros_row = jnp.zeros((RPT,), jnp.float32)

    degp = _deg_partials(ei, ones_row, zeros_row)
    g0 = _tc_first(xp, Wg0, degp)
    p0 = _edge_scatter(ei, g0, zeros_tile)
    g1 = _tc_mid(p0, g0, degp, bg0, Wg1)
    p1 = _edge_scatter(ei, g1, zeros_tile)
    return _tc_tail(p1, g1, degp, bg1, batch_p,
                    Wf0, bf0, gamma0, beta0, Wf1, bf1, gamma1, beta1)
